# baseline (speedup 1.0000x reference)
import jax
import jax.numpy as jnp
from jax.experimental import pallas as pl
from jax.experimental.pallas import tpu as pltpu

D = 2048
N_TOK = 32768
B = 16
R = 2048
NR = 2
RS = R // NR
RS2 = 128
NEDGE = 10
NB = N_TOK // R
CW = 1024
NJ = D // CW
NEG = -1e30


def _min_st(idx_ref):
    m = idx_ref[0, 0]
    for b in range(1, B):
        m = jnp.minimum(m, idx_ref[b, 0])
    return m


def _max_en(idx_ref):
    m = idx_ref[0, 1]
    for b in range(1, B):
        m = jnp.maximum(m, idx_ref[b, 1])
    return m


def _main_bounds(idx_ref):
    mn, mx = _min_st(idx_ref), _max_en(idx_ref)
    b_lo_m = (mn + RS - 1) // RS
    b_hi_m = (mx + 1) // RS - 1
    return mn, mx, b_lo_m, b_hi_m


def _front_blocks(idx_ref):
    mn, mx, b_lo_m, _ = _main_bounds(idx_ref)
    first = mn // RS2
    last = jnp.minimum(b_lo_m * RS - 1, mx) // RS2
    return first, jnp.maximum(last, first)


def _back_blocks(idx_ref):
    mn, mx, b_lo_m, b_hi_m = _main_bounds(idx_ref)
    bound = jnp.maximum((b_hi_m + 1) * RS, b_lo_m * RS)
    first = bound // RS2
    last = jnp.maximum(mx // RS2, first)
    return first, last


def _body(sidx_ref, idx_ref, watt_ref, batt_ref, wout_ref, bout_ref, *refs):
    a_refs = refs[:NR]
    fr_ref, bk_ref = refs[NR], refs[NR + 1]
    out_ref, m_ref, l_ref, acc_ref = refs[NR + 2:NR + 6]
    i = pl.program_id(0)
    mn, mx, b_lo_m, b_hi_m = _main_bounds(sidx_ref)
    span_lo = mn // RS
    span_hi = mx // RS

    @pl.when(i == 0)
    def _init():
        m_ref[...] = jnp.full_like(m_ref, NEG)
        l_ref[...] = jnp.zeros_like(l_ref)
        acc_ref[...] = jnp.zeros_like(acc_ref)

    @pl.when((span_lo + i * NR <= span_hi) | (i < NEDGE))
    def _pool():
        w = watt_ref[...]
        st = idx_ref[...][:, 0][None, :]
        en = idx_ref[...][:, 1][None, :]

        f_first, _ = _front_blocks(sidx_ref)
        bk_first, _ = _back_blocks(sidx_ref)
        main_hi = (b_hi_m + 1) * RS
        streams = [
            (a_refs[k], (b_lo_m + i * NR + k) * RS, None, main_hi)
            for k in range(NR)
        ] + [
            (fr_ref, (f_first + i) * RS2, None, b_lo_m * RS),
            (bk_ref, (bk_first + i) * RS2,
             jnp.maximum((b_hi_m + 1) * RS, b_lo_m * RS), None),
        ]

        sbs, avals = [], []
        for ref, base, glo, ghi in streams:
            a = ref[...]
            n = a.shape[0]
            s = jax.lax.dot_general(
                a, w, (((1,), (0,)), ((), ())),
                preferred_element_type=jnp.float32) + batt_ref[0, 0]
            pos = base + jax.lax.broadcasted_iota(jnp.int32, (n, B), 0)
            mask = (pos >= st) & (pos <= en)
            if glo is not None:
                mask &= pos >= glo
            if ghi is not None:
                mask &= pos < ghi
            sbs.append(jnp.where(mask, s, NEG))
            avals.append(a)

        bm = sbs[0].max(axis=0)
        for sb in sbs[1:]:
            bm = jnp.maximum(bm, sb.max(axis=0))
        m_old = m_ref[0]
        m_new = jnp.maximum(m_old, bm)
        alpha = jnp.exp(m_old - m_new)
        es = [jnp.exp(sb - m_new[None, :]) for sb in sbs]
        lsum = es[0].sum(axis=0)
        for e in es[1:]:
            lsum = lsum + e.sum(axis=0)
        l_ref[0] = alpha * l_ref[0] + lsum
        m_ref[0] = m_new
        upd = jax.lax.dot_general(es[0], avals[0], (((0,), (0,)), ((), ())),
                                  preferred_element_type=jnp.float32)
        for e, a in zip(es[1:], avals[1:]):
            upd = upd + jax.lax.dot_general(
                e, a, (((0,), (0,)), ((), ())),
                preferred_element_type=jnp.float32)
        acc_ref[...] = acc_ref[...] * alpha[:, None] + upd

    @pl.when(i >= NB)
    def _project():
        pooled = acc_ref[...] / l_ref[0][:, None]
        out_ref[...] = jax.lax.dot_general(
            pooled, wout_ref[...], (((1,), (0,)), ((), ())),
            preferred_element_type=jnp.float32) + bout_ref[...]


def _a_spec(k):
    def imap(i, sidx_ref):
        _, _, b_lo_m, b_hi_m = _main_bounds(sidx_ref)
        v = b_lo_m + i * NR + k
        return (jnp.clip(v, b_lo_m, jnp.maximum(b_hi_m, b_lo_m)), 0)
    return pl.BlockSpec((RS, D), imap)


def _front_spec():
    def imap(i, sidx_ref):
        first, last = _front_blocks(sidx_ref)
        return (jnp.minimum(first + i, last), 0)
    return pl.BlockSpec((RS2, D), imap)


def _back_spec():
    def imap(i, sidx_ref):
        first, last = _back_blocks(sidx_ref)
        return (jnp.minimum(first + i, last), 0)
    return pl.BlockSpec((RS2, D), imap)


def _jmap(i, sidx_ref):
    del sidx_ref
    return (0, jnp.maximum(i - NB, 0))


@jax.jit
def kernel(atom_features, index_list, W_att, b_att, W_out, b_out):
    idx32 = index_list.astype(jnp.int32)
    return pl.pallas_call(
        _body,
        grid_spec=pltpu.PrefetchScalarGridSpec(
            num_scalar_prefetch=1,
            grid=(NB + NJ,),
            in_specs=[
                pl.BlockSpec((B, 2), lambda i, s: (0, 0)),
                pl.BlockSpec((D, 1), lambda i, s: (0, 0)),
                pl.BlockSpec((1, 1), lambda i, s: (0, 0)),
                pl.BlockSpec((D, CW), _jmap),
                pl.BlockSpec((1, CW), _jmap),
            ] + [_a_spec(k) for k in range(NR)]
              + [_front_spec(), _back_spec()],
            out_specs=pl.BlockSpec((B, CW), _jmap),
            scratch_shapes=[
                pltpu.VMEM((1, B), jnp.float32),
                pltpu.VMEM((1, B), jnp.float32),
                pltpu.VMEM((B, D), jnp.float32),
            ],
        ),
        out_shape=jax.ShapeDtypeStruct((B, D), jnp.float32),
    )(idx32, idx32, W_att, b_att.reshape(1, 1), W_out, b_out.reshape(1, D),
      *([atom_features] * (NR + 2)))


# speedup vs baseline: 1.0673x; 1.0673x over previous
import jax
import jax.numpy as jnp
from jax.experimental import pallas as pl
from jax.experimental.pallas import tpu as pltpu

D = 2048
N_TOK = 32768
B = 16
R = 2048
NR = 2
RS = R // NR
NB = N_TOK // R
CW = 1024
NJ = D // CW
NEG = -1e30


def _first_sub(idx_ref):
    m = idx_ref[0, 0]
    for b in range(1, B):
        m = jnp.minimum(m, idx_ref[b, 0])
    return m // RS


def _last_sub(idx_ref):
    m = idx_ref[0, 1]
    for b in range(1, B):
        m = jnp.maximum(m, idx_ref[b, 1])
    return m // RS


def _body(sidx_ref, idx_ref, watt_ref, batt_ref, wout_ref, bout_ref, *refs):
    a_refs = refs[:NR]
    out_ref, m_ref, l_ref, acc_ref = refs[NR:NR + 4]
    i = pl.program_id(0)
    b_lo = _first_sub(sidx_ref)
    b_hi = _last_sub(sidx_ref)

    @pl.when(i == 0)
    def _init():
        m_ref[...] = jnp.full_like(m_ref, NEG)
        l_ref[...] = jnp.zeros_like(l_ref)
        acc_ref[...] = jnp.zeros_like(acc_ref)

    @pl.when(b_lo + i * NR <= b_hi)
    def _pool():
        a = [r[...] for r in a_refs]
        w = watt_ref[...]
        st = idx_ref[...][:, 0][None, :]
        en = idx_ref[...][:, 1][None, :]

        sbs = []
        for k in range(NR):
            s = jax.lax.dot_general(
                a[k], w, (((1,), (0,)), ((), ())),
                preferred_element_type=jnp.float32) + batt_ref[0, 0]
            pos = (b_lo + i * NR + k) * RS + jax.lax.broadcasted_iota(
                jnp.int32, (RS, B), 0)
            mask = (pos >= st) & (pos <= en)
            sbs.append(jnp.where(mask, s, NEG))

        bm = sbs[0].max(axis=0)
        for k in range(1, NR):
            bm = jnp.maximum(bm, sbs[k].max(axis=0))
        m_old = m_ref[0]
        m_new = jnp.maximum(m_old, bm)
        alpha = jnp.exp(m_old - m_new)
        es = [jnp.exp(sb - m_new[None, :]) for sb in sbs]
        lsum = es[0].sum(axis=0)
        for k in range(1, NR):
            lsum = lsum + es[k].sum(axis=0)
        l_ref[0] = alpha * l_ref[0] + lsum
        m_ref[0] = m_new
        upd = jax.lax.dot_general(es[0], a[0], (((0,), (0,)), ((), ())),
                                  preferred_element_type=jnp.float32)
        for k in range(1, NR):
            upd = upd + jax.lax.dot_general(
                es[k], a[k], (((0,), (0,)), ((), ())),
                preferred_element_type=jnp.float32)
        acc_ref[...] = acc_ref[...] * alpha[:, None] + upd

    @pl.when(i >= NB)
    def _project():
        pooled = acc_ref[...] / l_ref[0][:, None]
        out_ref[...] = jax.lax.dot_general(
            pooled, wout_ref[...], (((1,), (0,)), ((), ())),
            preferred_element_type=jnp.float32) + bout_ref[...]


def _a_spec(k):
    def imap(i, sidx_ref):
        v = _first_sub(sidx_ref) + i * NR + k
        return (jnp.minimum(v, _last_sub(sidx_ref)), 0)
    return pl.BlockSpec((RS, D), imap)


def _jmap(i, sidx_ref):
    del sidx_ref
    return (0, jnp.maximum(i - NB, 0))


@jax.jit
def kernel(atom_features, index_list, W_att, b_att, W_out, b_out):
    idx32 = index_list.astype(jnp.int32)
    return pl.pallas_call(
        _body,
        grid_spec=pltpu.PrefetchScalarGridSpec(
            num_scalar_prefetch=1,
            grid=(NB + NJ,),
            in_specs=[
                pl.BlockSpec((B, 2), lambda i, s: (0, 0)),
                pl.BlockSpec((D, 1), lambda i, s: (0, 0)),
                pl.BlockSpec((1, 1), lambda i, s: (0, 0)),
                pl.BlockSpec((D, CW), _jmap),
                pl.BlockSpec((1, CW), _jmap),
            ] + [_a_spec(k) for k in range(NR)],
            out_specs=pl.BlockSpec((B, CW), _jmap),
            scratch_shapes=[
                pltpu.VMEM((1, B), jnp.float32),
                pltpu.VMEM((1, B), jnp.float32),
                pltpu.VMEM((B, D), jnp.float32),
            ],
        ),
        out_shape=jax.ShapeDtypeStruct((B, D), jnp.float32),
    )(idx32, idx32, W_att, b_att.reshape(1, 1), W_out, b_out.reshape(1, D),
      *([atom_features] * NR))
